# baseline (speedup 1.0000x reference)
import jax
import jax.numpy as jnp
from jax.experimental import pallas as pl
from jax.experimental.pallas import tpu as pltpu

_EPS = 1e-8
_HALF = 4096
_N = 8192
_D = 256
_BR = 512
_NB = _N // _BR
_NBH = _HALF // _BR
_LT = _BR // 128
_C = 2.8853900817779268
_SQRT_C = 1.6986436287041668


def _blk(zn_ref, b):
    return zn_ref[b // _NBH, pl.ds((b % _NBH) * _BR, _BR), :]


def _emit(zn_ref, rsacc_ref, colacc_ref, out_ref, m):
    rows_f = _blk(zn_ref, m).astype(jnp.float32)
    pair_f = _blk(zn_ref, (m + _NBH) % _NB).astype(jnp.float32)
    pos_c = jnp.sum(rows_f * pair_f, axis=1, keepdims=True)
    self_c = jnp.sum(rows_f * rows_f, axis=1, keepdims=True)
    total = (jnp.sum(rsacc_ref[m], axis=1, keepdims=True)
             + jnp.transpose(colacc_ref[m], (1, 0)))
    out_ref[pl.ds(m * _BR, _BR), :] = (
        jnp.log(total - jnp.exp2(self_c)) - (2.0 / _C) * pos_c)


def _ntxent_kernel(zi_ref, zj_ref, out_ref, zn_ref, rsacc_ref, colacc_ref):
    i = pl.program_id(0)

    @pl.when(i == 0)
    def _zero():
        rsacc_ref[...] = jnp.zeros_like(rsacc_ref)
        colacc_ref[...] = jnp.zeros_like(colacc_ref)

    @pl.when(i < _NBH)
    def _normalize():
        for h, ref in enumerate((zi_ref, zj_ref)):
            z = ref[...]
            nrm = jnp.sqrt(jnp.sum(z * z, axis=1, keepdims=True))
            scl = _SQRT_C / jnp.maximum(nrm, _EPS)
            zn_ref[h, pl.ds(i * _BR, _BR), :] = (
                (z * scl).astype(jnp.bfloat16))

    @pl.when(i >= _NBH)
    def _compute():
        k = i - _NBH
        ra = k
        rbb = _NB - 1 - k
        for t in range(_NB + 1):
            is_a = ra + t <= _NB - 1
            rb = jnp.where(is_a, ra, rbb)
            cb = jnp.where(is_a, ra + t, t - 1)
            s = jax.lax.dot_general(
                _blk(zn_ref, rb), _blk(zn_ref, cb),
                (((1,), (1,)), ((), ())),
                preferred_element_type=jnp.float32)
            e = jnp.exp2(s)
            part = e[:, 0:128]
            for lt in range(1, _LT):
                part = part + e[:, lt * 128:(lt + 1) * 128]
            rsacc_ref[rb] += part
            credit = jnp.where(cb > rb, 1.0, 0.0)
            colacc_ref[cb] += credit * jnp.sum(e, axis=0, keepdims=True)

        _emit(zn_ref, rsacc_ref, colacc_ref, out_ref, ra)

        @pl.when(k == _NB // 2 - 1)
        def _emit_tail():
            for m in range(_NB // 2, _NB):
                _emit(zn_ref, rsacc_ref, colacc_ref, out_ref, m)


def kernel(z_i, z_j):
    per_row = pl.pallas_call(
        _ntxent_kernel,
        out_shape=jax.ShapeDtypeStruct((_N, 1), jnp.float32),
        grid=(_NBH + _NB // 2,),
        in_specs=[
            pl.BlockSpec((_BR, _D),
                         lambda i: (jnp.minimum(i, _NBH - 1), 0)),
            pl.BlockSpec((_BR, _D),
                         lambda i: (jnp.minimum(i, _NBH - 1), 0)),
        ],
        out_specs=pl.BlockSpec((_N, 1), lambda i: (0, 0)),
        scratch_shapes=[
            pltpu.VMEM((2, _HALF, _D), jnp.bfloat16),
            pltpu.VMEM((_NB, _BR, 128), jnp.float32),
            pltpu.VMEM((_NB, 1, _BR), jnp.float32),
        ],
        compiler_params=pltpu.CompilerParams(
            dimension_semantics=("arbitrary",),
            vmem_limit_bytes=50 * 1024 * 1024),
        name="ntxent_loss",
    )(z_i, z_j)
    return jnp.mean(per_row)


# speedup vs baseline: 1.1358x; 1.1358x over previous
import jax
import jax.numpy as jnp
from jax.experimental import pallas as pl
from jax.experimental.pallas import tpu as pltpu

_EPS = 1e-8
_HALF = 4096
_N = 8192
_D = 256
_BR = 512
_NB = _N // _BR
_NBH = _HALF // _BR
_LT = _BR // 128
_NORM_BLK = 512
_C = 2.8853900817779268
_SQRT_C = 1.6986436287041668


def _blk(zn_ref, b):
    return zn_ref[b // _NBH, pl.ds((b % _NBH) * _BR, _BR), :]


def _emit(zn_ref, rsacc_ref, colacc_ref, out_ref, m):
    rows_f = _blk(zn_ref, m).astype(jnp.float32)
    pair_f = _blk(zn_ref, (m + _NBH) % _NB).astype(jnp.float32)
    pos_c = jnp.sum(rows_f * pair_f, axis=1, keepdims=True)
    self_c = jnp.sum(rows_f * rows_f, axis=1, keepdims=True)
    total = (jnp.sum(rsacc_ref[m], axis=1, keepdims=True)
             + jnp.transpose(colacc_ref[m], (1, 0)))
    vals = jnp.log(total - jnp.exp2(self_c)) - (2.0 / _C) * pos_c
    out_ref[0, 0] += jnp.sum(vals)


def _ntxent_kernel(zi_ref, zj_ref, out_ref, zn_ref, rsacc_ref, colacc_ref):
    i = pl.program_id(0)

    @pl.when(i == 0)
    def _init():
        for h, ref in enumerate((zi_ref, zj_ref)):
            for k in range(_HALF // _NORM_BLK):
                z = ref[k * _NORM_BLK:(k + 1) * _NORM_BLK, :]
                nrm = jnp.sqrt(jnp.sum(z * z, axis=1, keepdims=True))
                scl = _SQRT_C / jnp.maximum(nrm, _EPS)
                zn_ref[h, k * _NORM_BLK:(k + 1) * _NORM_BLK, :] = (
                    (z * scl).astype(jnp.bfloat16))
        rsacc_ref[...] = jnp.zeros_like(rsacc_ref)
        colacc_ref[...] = jnp.zeros_like(colacc_ref)
        out_ref[0, 0] = 0.0

    ra = i
    rbb = _NB - 1 - i
    for t in range(_NB + 1):
        is_a = ra + t <= _NB - 1
        rb = jnp.where(is_a, ra, rbb)
        cb = jnp.where(is_a, ra + t, t - 1)
        s = jax.lax.dot_general(
            _blk(zn_ref, rb), _blk(zn_ref, cb), (((1,), (1,)), ((), ())),
            preferred_element_type=jnp.float32)
        e = jnp.exp2(s)
        part = e[:, 0:128]
        for lt in range(1, _LT):
            part = part + e[:, lt * 128:(lt + 1) * 128]
        rsacc_ref[rb] += part
        credit = jnp.where(cb > rb, 1.0, 0.0)
        colacc_ref[cb] += credit * jnp.sum(e, axis=0, keepdims=True)

    _emit(zn_ref, rsacc_ref, colacc_ref, out_ref, ra)

    @pl.when(i == _NB // 2 - 1)
    def _finish_tail():
        for m in range(_NB // 2, _NB):
            _emit(zn_ref, rsacc_ref, colacc_ref, out_ref, m)
        out_ref[0, 0] = out_ref[0, 0] * (1.0 / _N)


def kernel(z_i, z_j):
    loss = pl.pallas_call(
        _ntxent_kernel,
        out_shape=jax.ShapeDtypeStruct((1, 1), jnp.float32),
        grid=(_NB // 2,),
        in_specs=[
            pl.BlockSpec((_HALF, _D), lambda i: (0, 0)),
            pl.BlockSpec((_HALF, _D), lambda i: (0, 0)),
        ],
        out_specs=pl.BlockSpec(memory_space=pltpu.SMEM),
        scratch_shapes=[
            pltpu.VMEM((2, _HALF, _D), jnp.bfloat16),
            pltpu.VMEM((_NB, _BR, 128), jnp.float32),
            pltpu.VMEM((_NB, 1, _BR), jnp.float32),
        ],
        compiler_params=pltpu.CompilerParams(
            dimension_semantics=("arbitrary",),
            vmem_limit_bytes=50 * 1024 * 1024),
        name="ntxent_loss",
    )(z_i, z_j)
    return jnp.reshape(loss, ())
